# Initial kernel scaffold; baseline (speedup 1.0000x reference)
import jax
import jax.numpy as jnp
from jax.experimental import pallas as pl


def kernel(x, token_table, pos_table):
    raise NotImplementedError("write your pallas kernel here")


# speedup vs baseline: 3.0709x; 3.0709x over previous
import functools

import jax
import jax.numpy as jnp
from jax import lax
from jax.experimental import pallas as pl
from jax.experimental.pallas import tpu as pltpu
from jax.experimental.pallas import tpu_sc as plsc

_NC = 2
_NS = 16
_HALF = 100


def kernel(x, token_table, pos_table):
    B, S = x.shape
    V, E = token_table.shape
    nw = _NC * _NS
    seq_per_w = B // nw
    n_half = S // _HALF

    x3 = x.astype(jnp.int32).reshape(B, n_half, _HALF)

    mesh = plsc.VectorSubcoreMesh(core_axis_name="c", subcore_axis_name="s")

    @functools.partial(
        pl.kernel,
        mesh=mesh,
        out_type=jax.ShapeDtypeStruct((B, S, E), jnp.float32),
        scratch_types=[
            pltpu.VMEM((n_half, _HALF), jnp.int32),
            pltpu.VMEM((S, E), jnp.float32),
            pltpu.VMEM((S, E), jnp.float32),
            pltpu.SemaphoreType.DMA,
        ],
        compiler_params=pltpu.CompilerParams(use_tc_tiling_on_sc=False),
    )
    def emb_kernel(x_hbm, tok_hbm, pos_hbm, out_hbm, idx_v, rows_v, pos_v, sem):
        wid = lax.axis_index("s") * _NC + lax.axis_index("c")
        pltpu.sync_copy(pos_hbm, pos_v)

        def body(g, carry):
            s = wid * seq_per_w + g
            pltpu.sync_copy(x_hbm.at[s], idx_v)
            cps = [
                pltpu.async_copy(
                    tok_hbm.at[idx_v.at[h]],
                    rows_v.at[pl.ds(h * _HALF, _HALF)],
                    sem,
                )
                for h in range(n_half)
            ]
            for cp in cps:
                cp.wait()

            def add_row(i, c):
                for d in range(E // 16):
                    sl = pl.ds(d * 16, 16)
                    rows_v[i, sl] = rows_v[i, sl] + pos_v[i, sl]
                return c

            lax.fori_loop(0, S, add_row, 0)
            pltpu.sync_copy(rows_v, out_hbm.at[s])
            return carry

        lax.fori_loop(0, seq_per_w, body, 0)

    return emb_kernel(x3, token_table, pos_table)


# speedup vs baseline: 4.0507x; 1.3191x over previous
import functools

import jax
import jax.numpy as jnp
from jax import lax
from jax.experimental import pallas as pl
from jax.experimental.pallas import tpu as pltpu
from jax.experimental.pallas import tpu_sc as plsc

_NC = 2
_NS = 16
_HALF = 100


def kernel(x, token_table, pos_table):
    B, S = x.shape
    V, E = token_table.shape
    nw = _NC * _NS
    n_seq = B // nw
    n_half = S // _HALF

    x4 = x.astype(jnp.int32).reshape(nw, n_seq, n_half, _HALF)

    mesh = plsc.VectorSubcoreMesh(core_axis_name="c", subcore_axis_name="s")

    @functools.partial(
        pl.kernel,
        mesh=mesh,
        out_type=jax.ShapeDtypeStruct((B, S, E), jnp.float32),
        scratch_types=[
            pltpu.VMEM((n_seq, n_half, _HALF), jnp.int32),
            pltpu.VMEM((2, S, E), jnp.float32),
            pltpu.VMEM((2, S, E), jnp.float32),
            pltpu.VMEM((S, E), jnp.float32),
            pltpu.SemaphoreType.DMA,
            pltpu.SemaphoreType.DMA,
            pltpu.SemaphoreType.DMA,
            pltpu.SemaphoreType.DMA,
        ],
        compiler_params=pltpu.CompilerParams(use_tc_tiling_on_sc=False),
    )
    def emb_kernel(x_hbm, tok_hbm, pos_hbm, out_hbm, idx_v, rows_v, obuf_v,
                   pos_v, gsem0, gsem1, osem0, osem1):
        wid = lax.axis_index("s") * _NC + lax.axis_index("c")
        base = wid * n_seq
        gsem = (gsem0, gsem1)
        osem = (osem0, osem1)

        pltpu.sync_copy(pos_hbm, pos_v)
        pltpu.sync_copy(x_hbm.at[wid], idx_v)

        def gather_start(g, b):
            for h in range(n_half):
                pltpu.async_copy(
                    tok_hbm.at[idx_v.at[g, h]],
                    rows_v.at[b, pl.ds(h * _HALF, _HALF)],
                    gsem[b],
                )

        def gather_wait(b):
            for h in range(n_half):
                pltpu.make_async_copy(
                    tok_hbm.at[idx_v.at[0, h]],
                    rows_v.at[b, pl.ds(h * _HALF, _HALF)],
                    gsem[b],
                ).wait()

        def add_pos(b):
            def add_row(i, c):
                for r in range(4):
                    row = i * 4 + r
                    for d in range(E // 16):
                        sl = pl.ds(d * 16, 16)
                        obuf_v[b, row, sl] = rows_v[b, row, sl] + pos_v[row, sl]
                return c

            lax.fori_loop(0, S // 4, add_row, 0)

        def out_start(g, b):
            pltpu.async_copy(obuf_v.at[b], out_hbm.at[base + g], osem[b])

        def out_wait(b):
            pltpu.make_async_copy(obuf_v.at[b], out_hbm.at[base], osem[b]).wait()

        gather_start(0, 0)
        gather_start(1, 1)
        for g in range(2):
            gather_wait(g)
            add_pos(g)
            gather_start(g + 2, g)
            out_start(g, g)

        def body(gg, carry):
            g0 = gg * 2
            for b in range(2):
                g = g0 + b
                gather_wait(b)
                out_wait(b)
                add_pos(b)
                gather_start(g + 2, b)
                out_start(g, b)
            return carry

        lax.fori_loop(1, n_seq // 2 - 1, body, 0)

        for g in range(n_seq - 2, n_seq):
            b = g % 2
            gather_wait(b)
            out_wait(b)
            add_pos(b)
            out_start(g, b)
        out_wait(0)
        out_wait(1)

    return emb_kernel(x4, token_table, pos_table)
